# baseline (speedup 1.0000x reference)
import functools

import jax
import jax.numpy as jnp
from jax.experimental import pallas as pl

ROW_BLOCK = 256


def _topk_threshold_bits(h, k):
    hb = jax.lax.bitcast_convert_type(h, jnp.int32)
    rows = h.shape[0]
    t0 = jnp.zeros((rows, 1), dtype=jnp.int32)
    ones = jnp.ones((h.shape[1], 8), dtype=jnp.float32)

    def body(i, t):
        bit = 30 - i
        cand = t | (jnp.int32(1) << bit)
        mask_f = jnp.where(hb >= cand, 1.0, 0.0).astype(jnp.float32)
        cnt = jnp.dot(mask_f, ones, preferred_element_type=jnp.float32)[:, :1]
        return jnp.where(cnt >= k, cand, t)

    t = jax.lax.fori_loop(0, 31, body, t0, unroll=True)
    return hb, t


def _encode_t(a, w):
    return jax.lax.dot_general(
        a, w, (((1,), (1,)), ((), ())), preferred_element_type=jnp.float32
    )


def _level0_kernel(x_ref, wenc_ref, benc_ref, wdec_ref, bdec_ref, o_ref, *, k):
    xb = x_ref[...] - bdec_ref[...]
    h = jnp.maximum(_encode_t(xb, wenc_ref[...]) + benc_ref[...], 0.0)
    hb, t = _topk_threshold_bits(h, k)
    z = jnp.where(hb >= t, h, 0.0)
    o_ref[...] = (
        jnp.dot(z, wdec_ref[...], preferred_element_type=jnp.float32)
        + bdec_ref[...]
    )


def _level1_kernel(rb_ref, wenc_ref, benc_ref, wdec_ref, bdec_ref, o_ref, *, k,
                   w0, w1):
    rb = rb_ref[...]
    h = jnp.maximum(_encode_t(rb, wenc_ref[...]) + benc_ref[...], 0.0)
    hb, t = _topk_threshold_bits(h, k)
    z = jnp.where(hb >= t, h, 0.0)
    recon1 = (
        jnp.dot(z, wdec_ref[...], preferred_element_type=jnp.float32)
        + bdec_ref[...]
    )
    o_ref[...] = w0 * rb + w1 * recon1


def kernel(x, W_enc, b_enc, W_dec, b_dec, W_enc1, b_enc1, W_dec1, b_dec1):
    n, d_in = x.shape
    hidden = W_enc.shape[0]
    level = W_enc1.shape[0]
    nb = n // ROW_BLOCK

    benc2 = b_enc.reshape(1, hidden)
    bdec2 = b_dec.reshape(1, d_in)
    benc12 = b_enc1.reshape(1, level)
    bdec12 = b_dec1.reshape(1, d_in)

    recon_base = pl.pallas_call(
        functools.partial(_level0_kernel, k=64),
        grid=(nb,),
        in_specs=[
            pl.BlockSpec((ROW_BLOCK, d_in), lambda i: (i, 0)),
            pl.BlockSpec((hidden, d_in), lambda i: (0, 0)),
            pl.BlockSpec((1, hidden), lambda i: (0, 0)),
            pl.BlockSpec((hidden, d_in), lambda i: (0, 0)),
            pl.BlockSpec((1, d_in), lambda i: (0, 0)),
        ],
        out_specs=pl.BlockSpec((ROW_BLOCK, d_in), lambda i: (i, 0)),
        out_shape=jax.ShapeDtypeStruct((n, d_in), jnp.float32),
    )(x, W_enc, benc2, W_dec, bdec2)

    out = pl.pallas_call(
        functools.partial(
            _level1_kernel, k=32, w0=float(2.0 / 3.0), w1=float(1.0 / 3.0)
        ),
        grid=(nb,),
        in_specs=[
            pl.BlockSpec((ROW_BLOCK, d_in), lambda i: (i, 0)),
            pl.BlockSpec((level, d_in), lambda i: (0, 0)),
            pl.BlockSpec((1, level), lambda i: (0, 0)),
            pl.BlockSpec((level, d_in), lambda i: (0, 0)),
            pl.BlockSpec((1, d_in), lambda i: (0, 0)),
        ],
        out_specs=pl.BlockSpec((ROW_BLOCK, d_in), lambda i: (i, 0)),
        out_shape=jax.ShapeDtypeStruct((n, d_in), jnp.float32),
    )(recon_base, W_enc1, benc12, W_dec1, bdec12)

    return out


# speedup vs baseline: 2.8394x; 2.8394x over previous
import functools

import jax
import jax.numpy as jnp
from jax.experimental import pallas as pl

ROW_BLOCK = 256


def _topk_threshold_bits(h, k):
    rows = h.shape[0]
    m = jnp.max(h, axis=1, keepdims=True)
    scale = 8388608.0 / jnp.maximum(m, 1e-30)
    q = (h * scale).astype(jnp.int32)
    t0 = jnp.zeros((rows, 1), dtype=jnp.int32)

    def body(i, t):
        bit = 24 - i
        cand = t | (jnp.int32(1) << bit)
        cnt = jnp.sum((q >= cand).astype(jnp.float32), axis=1, keepdims=True)
        return jnp.where(cnt >= k, cand, t)

    t = jax.lax.fori_loop(0, 25, body, t0, unroll=True)
    return q, t


def _encode_t(a, w):
    return jax.lax.dot_general(
        a, w, (((1,), (1,)), ((), ())), preferred_element_type=jnp.float32
    )


def _level0_kernel(x_ref, wenc_ref, benc_ref, wdec_ref, bdec_ref, o_ref, *, k):
    xb = x_ref[...] - bdec_ref[...]
    h = jnp.maximum(_encode_t(xb, wenc_ref[...]) + benc_ref[...], 0.0)
    hb, t = _topk_threshold_bits(h, k)
    z = jnp.where(hb >= t, h, 0.0)
    o_ref[...] = (
        jnp.dot(z, wdec_ref[...], preferred_element_type=jnp.float32)
        + bdec_ref[...]
    )


def _level1_kernel(rb_ref, wenc_ref, benc_ref, wdec_ref, bdec_ref, o_ref, *, k,
                   w0, w1):
    rb = rb_ref[...]
    h = jnp.maximum(_encode_t(rb, wenc_ref[...]) + benc_ref[...], 0.0)
    hb, t = _topk_threshold_bits(h, k)
    z = jnp.where(hb >= t, h, 0.0)
    recon1 = (
        jnp.dot(z, wdec_ref[...], preferred_element_type=jnp.float32)
        + bdec_ref[...]
    )
    o_ref[...] = w0 * rb + w1 * recon1


def kernel(x, W_enc, b_enc, W_dec, b_dec, W_enc1, b_enc1, W_dec1, b_dec1):
    n, d_in = x.shape
    hidden = W_enc.shape[0]
    level = W_enc1.shape[0]
    nb = n // ROW_BLOCK

    benc2 = b_enc.reshape(1, hidden)
    bdec2 = b_dec.reshape(1, d_in)
    benc12 = b_enc1.reshape(1, level)
    bdec12 = b_dec1.reshape(1, d_in)

    recon_base = pl.pallas_call(
        functools.partial(_level0_kernel, k=64),
        grid=(nb,),
        in_specs=[
            pl.BlockSpec((ROW_BLOCK, d_in), lambda i: (i, 0)),
            pl.BlockSpec((hidden, d_in), lambda i: (0, 0)),
            pl.BlockSpec((1, hidden), lambda i: (0, 0)),
            pl.BlockSpec((hidden, d_in), lambda i: (0, 0)),
            pl.BlockSpec((1, d_in), lambda i: (0, 0)),
        ],
        out_specs=pl.BlockSpec((ROW_BLOCK, d_in), lambda i: (i, 0)),
        out_shape=jax.ShapeDtypeStruct((n, d_in), jnp.float32),
    )(x, W_enc, benc2, W_dec, bdec2)

    out = pl.pallas_call(
        functools.partial(
            _level1_kernel, k=32, w0=float(2.0 / 3.0), w1=float(1.0 / 3.0)
        ),
        grid=(nb,),
        in_specs=[
            pl.BlockSpec((ROW_BLOCK, d_in), lambda i: (i, 0)),
            pl.BlockSpec((level, d_in), lambda i: (0, 0)),
            pl.BlockSpec((1, level), lambda i: (0, 0)),
            pl.BlockSpec((level, d_in), lambda i: (0, 0)),
            pl.BlockSpec((1, d_in), lambda i: (0, 0)),
        ],
        out_specs=pl.BlockSpec((ROW_BLOCK, d_in), lambda i: (i, 0)),
        out_shape=jax.ShapeDtypeStruct((n, d_in), jnp.float32),
    )(recon_base, W_enc1, benc12, W_dec1, bdec12)

    return out


# speedup vs baseline: 2.8990x; 1.0210x over previous
import functools

import jax
import jax.numpy as jnp
from jax.experimental import pallas as pl

ROW_BLOCK = 256
ROW_BLOCK1 = 512


def _topk_threshold_bits(h, k):
    rows = h.shape[0]
    m = jnp.max(h, axis=1, keepdims=True)
    scale = 8388608.0 / jnp.maximum(m, 1e-30)
    q = (h * scale).astype(jnp.int32)
    t0 = jnp.zeros((rows, 1), dtype=jnp.int32)

    def body(i, t):
        bit = 23 - i
        cand = t | (jnp.int32(1) << bit)
        cnt = jnp.sum((q >= cand).astype(jnp.float32), axis=1, keepdims=True)
        return jnp.where(cnt >= k, cand, t)

    t = jax.lax.fori_loop(0, 24, body, t0, unroll=True)
    return q, t


def _encode_t(a, w):
    return jax.lax.dot_general(
        a, w, (((1,), (1,)), ((), ())), preferred_element_type=jnp.float32
    )


def _level0_kernel(x_ref, wenc_ref, benc_ref, wdec_ref, bdec_ref, o_ref, *, k):
    xb = x_ref[...] - bdec_ref[...]
    h = jnp.maximum(_encode_t(xb, wenc_ref[...]) + benc_ref[...], 0.0)
    hb, t = _topk_threshold_bits(h, k)
    z = jnp.where(hb >= t, h, 0.0)
    o_ref[...] = (
        jnp.dot(z, wdec_ref[...], preferred_element_type=jnp.float32)
        + bdec_ref[...]
    )


def _level1_kernel(rb_ref, wenc_ref, benc_ref, wdec_ref, bdec_ref, o_ref, *, k,
                   w0, w1):
    rb = rb_ref[...]
    h = jnp.maximum(_encode_t(rb, wenc_ref[...]) + benc_ref[...], 0.0)
    hb, t = _topk_threshold_bits(h, k)
    z = jnp.where(hb >= t, h, 0.0)
    recon1 = (
        jnp.dot(z, wdec_ref[...], preferred_element_type=jnp.float32)
        + bdec_ref[...]
    )
    o_ref[...] = w0 * rb + w1 * recon1


def kernel(x, W_enc, b_enc, W_dec, b_dec, W_enc1, b_enc1, W_dec1, b_dec1):
    n, d_in = x.shape
    hidden = W_enc.shape[0]
    level = W_enc1.shape[0]
    nb = n // ROW_BLOCK

    benc2 = b_enc.reshape(1, hidden)
    bdec2 = b_dec.reshape(1, d_in)
    benc12 = b_enc1.reshape(1, level)
    bdec12 = b_dec1.reshape(1, d_in)

    recon_base = pl.pallas_call(
        functools.partial(_level0_kernel, k=64),
        grid=(nb,),
        in_specs=[
            pl.BlockSpec((ROW_BLOCK, d_in), lambda i: (i, 0)),
            pl.BlockSpec((hidden, d_in), lambda i: (0, 0)),
            pl.BlockSpec((1, hidden), lambda i: (0, 0)),
            pl.BlockSpec((hidden, d_in), lambda i: (0, 0)),
            pl.BlockSpec((1, d_in), lambda i: (0, 0)),
        ],
        out_specs=pl.BlockSpec((ROW_BLOCK, d_in), lambda i: (i, 0)),
        out_shape=jax.ShapeDtypeStruct((n, d_in), jnp.float32),
    )(x, W_enc, benc2, W_dec, bdec2)

    nb1 = n // ROW_BLOCK1
    out = pl.pallas_call(
        functools.partial(
            _level1_kernel, k=32, w0=float(2.0 / 3.0), w1=float(1.0 / 3.0)
        ),
        grid=(nb1,),
        in_specs=[
            pl.BlockSpec((ROW_BLOCK1, d_in), lambda i: (i, 0)),
            pl.BlockSpec((level, d_in), lambda i: (0, 0)),
            pl.BlockSpec((1, level), lambda i: (0, 0)),
            pl.BlockSpec((level, d_in), lambda i: (0, 0)),
            pl.BlockSpec((1, d_in), lambda i: (0, 0)),
        ],
        out_specs=pl.BlockSpec((ROW_BLOCK1, d_in), lambda i: (i, 0)),
        out_shape=jax.ShapeDtypeStruct((n, d_in), jnp.float32),
    )(recon_base, W_enc1, benc12, W_dec1, bdec12)

    return out


# speedup vs baseline: 2.9819x; 1.0286x over previous
import functools

import jax
import jax.numpy as jnp
from jax.experimental import pallas as pl

ROW_BLOCK = 256
ROW_BLOCK1 = 512


def _topk_threshold_bits(h, k):
    rows = h.shape[0]
    m = jnp.max(h, axis=1, keepdims=True)
    scale = 8388608.0 / jnp.maximum(m, 1e-30)
    q = (h * scale).astype(jnp.int32)
    t0 = jnp.zeros((rows, 1), dtype=jnp.int32)

    def body(i, t):
        bit = 23 - i
        cand = t | (jnp.int32(1) << bit)
        cnt = jnp.sum((q >= cand).astype(jnp.float32), axis=1, keepdims=True)
        return jnp.where(cnt >= k, cand, t)

    t = jax.lax.fori_loop(0, 24, body, t0, unroll=True)
    return q, t


def _encode_t(a, w):
    return jax.lax.dot_general(
        a, w, (((1,), (1,)), ((), ())), preferred_element_type=jnp.float32
    )


def _level0_kernel(x_ref, wenc_ref, benc_ref, wdec_ref, bdec_ref, o_ref, *, k):
    half = x_ref.shape[0] // 2
    wenc = wenc_ref[...]
    wdec = wdec_ref[...]
    benc = benc_ref[...]
    bdec = bdec_ref[...]
    xa = x_ref[:half, :] - bdec
    xb = x_ref[half:, :] - bdec
    ha = jnp.maximum(_encode_t(xa, wenc) + benc, 0.0)
    hb_ = jnp.maximum(_encode_t(xb, wenc) + benc, 0.0)
    qa, ta = _topk_threshold_bits(ha, k)
    za = jnp.where(qa >= ta, ha, 0.0)
    o_ref[:half, :] = (
        jnp.dot(za, wdec, preferred_element_type=jnp.float32) + bdec
    )
    qb, tb = _topk_threshold_bits(hb_, k)
    zb = jnp.where(qb >= tb, hb_, 0.0)
    o_ref[half:, :] = (
        jnp.dot(zb, wdec, preferred_element_type=jnp.float32) + bdec
    )


def _level1_kernel(rb_ref, wenc_ref, benc_ref, wdec_ref, bdec_ref, o_ref, *, k,
                   w0, w1):
    half = rb_ref.shape[0] // 2
    wenc = wenc_ref[...]
    wdec = wdec_ref[...]
    benc = benc_ref[...]
    bdec = bdec_ref[...]
    ra = rb_ref[:half, :]
    rb_ = rb_ref[half:, :]
    ha = jnp.maximum(_encode_t(ra, wenc) + benc, 0.0)
    hb_ = jnp.maximum(_encode_t(rb_, wenc) + benc, 0.0)
    qa, ta = _topk_threshold_bits(ha, k)
    za = jnp.where(qa >= ta, ha, 0.0)
    o_ref[:half, :] = w0 * ra + w1 * (
        jnp.dot(za, wdec, preferred_element_type=jnp.float32) + bdec
    )
    qb, tb = _topk_threshold_bits(hb_, k)
    zb = jnp.where(qb >= tb, hb_, 0.0)
    o_ref[half:, :] = w0 * rb_ + w1 * (
        jnp.dot(zb, wdec, preferred_element_type=jnp.float32) + bdec
    )


def kernel(x, W_enc, b_enc, W_dec, b_dec, W_enc1, b_enc1, W_dec1, b_dec1):
    n, d_in = x.shape
    hidden = W_enc.shape[0]
    level = W_enc1.shape[0]
    nb = n // ROW_BLOCK

    benc2 = b_enc.reshape(1, hidden)
    bdec2 = b_dec.reshape(1, d_in)
    benc12 = b_enc1.reshape(1, level)
    bdec12 = b_dec1.reshape(1, d_in)

    recon_base = pl.pallas_call(
        functools.partial(_level0_kernel, k=64),
        grid=(nb,),
        in_specs=[
            pl.BlockSpec((ROW_BLOCK, d_in), lambda i: (i, 0)),
            pl.BlockSpec((hidden, d_in), lambda i: (0, 0)),
            pl.BlockSpec((1, hidden), lambda i: (0, 0)),
            pl.BlockSpec((hidden, d_in), lambda i: (0, 0)),
            pl.BlockSpec((1, d_in), lambda i: (0, 0)),
        ],
        out_specs=pl.BlockSpec((ROW_BLOCK, d_in), lambda i: (i, 0)),
        out_shape=jax.ShapeDtypeStruct((n, d_in), jnp.float32),
    )(x, W_enc, benc2, W_dec, bdec2)

    nb1 = n // ROW_BLOCK1
    out = pl.pallas_call(
        functools.partial(
            _level1_kernel, k=32, w0=float(2.0 / 3.0), w1=float(1.0 / 3.0)
        ),
        grid=(nb1,),
        in_specs=[
            pl.BlockSpec((ROW_BLOCK1, d_in), lambda i: (i, 0)),
            pl.BlockSpec((level, d_in), lambda i: (0, 0)),
            pl.BlockSpec((1, level), lambda i: (0, 0)),
            pl.BlockSpec((level, d_in), lambda i: (0, 0)),
            pl.BlockSpec((1, d_in), lambda i: (0, 0)),
        ],
        out_specs=pl.BlockSpec((ROW_BLOCK1, d_in), lambda i: (i, 0)),
        out_shape=jax.ShapeDtypeStruct((n, d_in), jnp.float32),
    )(recon_base, W_enc1, benc12, W_dec1, bdec12)

    return out
